# Initial kernel scaffold; baseline (speedup 1.0000x reference)
import jax
import jax.numpy as jnp
from jax.experimental import pallas as pl


def kernel(x, edge_index, W1, b1, W2, b2):
    raise NotImplementedError("write your pallas kernel here")


# speedup vs baseline: 14.4874x; 14.4874x over previous
import functools
import jax
import jax.numpy as jnp
import numpy as np
from jax import lax
from jax.experimental import pallas as pl
from jax.experimental.pallas import tpu as pltpu
from jax.experimental.pallas import tpu_sc as plsc

N = 10000
E = 320000
D1 = 128
DO = 2
DP = 16

NC = 2
NS = 16
NW = NC * NS
CH = 128

E1 = E + N + 1776
NCHUNK1 = E1 // (NW * CH)
PADROW = N + 16

ED = E + 3584
NCHUNK = ED // (NW * CH)

NACC = 10112
RPA = NACC // NS
RB = 8

_lo = 32 * (np.arange(D1 // 2) // 16) + np.arange(D1 // 2) % 16
PERM = np.concatenate([_lo, _lo + 16])

_mesh = plsc.VectorSubcoreMesh(core_axis_name="c", subcore_axis_name="s")
_f32 = jnp.float32
_i32 = jnp.int32


def _zero_fill(ref, rows, width):
    z = jnp.zeros((16,), _f32)

    def body(r, carry):
        for cdx in range(width // 16):
            ref[r, pl.ds(cdx * 16, 16)] = z
        return carry

    lax.fori_loop(0, rows, body, 0)


@functools.partial(
    pl.kernel,
    out_type=jax.ShapeDtypeStruct((NC * NACC, DP), _f32),
    mesh=_mesh,
    scratch_types=[
        pltpu.VMEM((NCHUNK, CH), _i32),
        pltpu.VMEM((CH, D1), _f32),
        pltpu.VMEM((RB, D1), _f32),
        pltpu.VMEM((RB, DP), _f32),
        pltpu.VMEM_SHARED((NACC, D1), _f32),
    ],
)
def _sc_degree(dst_hbm, out_hbm, dstv, ones_v, zb, rbo, acc):
    c = lax.axis_index("c")
    s = lax.axis_index("s")
    wid = s * NC + c
    row0 = s * RPA

    one = jnp.ones((16,), _f32)

    def fill_ones(r, carry):
        for q in range(D1 // 16):
            ones_v[r, pl.ds(16 * q, 16)] = one
        return carry

    lax.fori_loop(0, CH, fill_ones, 0)
    _zero_fill(zb, RB, D1)

    def zero(k, carry):
        pltpu.sync_copy(
            zb, acc.at[pl.ds(pl.multiple_of(row0 + k * RB, 8), RB)])
        return carry

    lax.fori_loop(0, RPA // RB, zero, 0)
    pltpu.sync_copy(dst_hbm.at[wid], dstv)
    plsc.subcore_barrier()

    def body(j, carry):
        pltpu.sync_copy(ones_v, acc.at[dstv.at[j]], add=True)
        return carry

    lax.fori_loop(0, NCHUNK, body, 0)
    plsc.subcore_barrier()

    def readback(k, carry):
        roff = pl.multiple_of(row0 + k * RB, 8)
        ooff = pl.multiple_of(c * NACC + row0 + k * RB, 8)
        pltpu.sync_copy(acc.at[pl.ds(roff, RB)], zb)
        for r in range(RB):
            rbo[r, pl.ds(0, 16)] = zb[r, pl.ds(0, 16)]
        pltpu.sync_copy(rbo, out_hbm.at[pl.ds(ooff, RB)])
        return carry

    lax.fori_loop(0, RPA // RB, readback, 0)


@functools.partial(
    pl.kernel,
    out_type=jax.ShapeDtypeStruct((NC * N, D1 // 2), _i32),
    mesh=_mesh,
    scratch_types=[
        pltpu.VMEM((NCHUNK1, CH), _i32),
        pltpu.VMEM((NCHUNK1, CH), _i32),
        pltpu.VMEM((CH, D1), _f32),
        pltpu.VMEM((RB, D1), _f32),
        pltpu.VMEM((RB, D1), _f32),
        pltpu.VMEM((RB, D1 // 2), _i32),
        pltpu.SemaphoreType.DMA,
        pltpu.VMEM_SHARED((NACC, D1), _f32),
    ],
)
def _sc_agg1(src_hbm, dst_hbm, tab_hbm, out_hbm, srcv, dstv, buf, zb,
             rbf, rbb, sem, acc):
    c = lax.axis_index("c")
    s = lax.axis_index("s")
    wid = s * NC + c
    row0 = s * RPA

    _zero_fill(zb, RB, D1)

    def zero(k, carry):
        pltpu.sync_copy(
            zb, acc.at[pl.ds(pl.multiple_of(row0 + k * RB, 8), RB)])
        return carry

    lax.fori_loop(0, RPA // RB, zero, 0)
    pltpu.sync_copy(src_hbm.at[wid], srcv)
    pltpu.sync_copy(dst_hbm.at[wid], dstv)
    plsc.subcore_barrier()

    def body(j, carry):
        pltpu.async_copy(tab_hbm.at[srcv.at[j]], buf, sem).wait()
        pltpu.sync_copy(buf, acc.at[dstv.at[j]], add=True)
        return carry

    lax.fori_loop(0, NCHUNK1, body, 0)
    plsc.subcore_barrier()

    rnd = jnp.full((16,), 0x8000, _i32)
    hmask = jnp.full((16,), -65536, _i32)
    nrows = jnp.minimum(RPA, N - row0)

    def readback(k, carry):
        roff = pl.multiple_of(row0 + k * RB, 8)
        ooff = pl.multiple_of(c * N + row0 + k * RB, 8)
        pltpu.sync_copy(acc.at[pl.ds(roff, RB)], rbf)
        for r in range(RB):
            for q in range(D1 // 32):
                a = lax.bitcast_convert_type(rbf[r, pl.ds(32 * q, 16)], _i32)
                b = lax.bitcast_convert_type(
                    rbf[r, pl.ds(32 * q + 16, 16)], _i32)
                lo = lax.shift_right_logical(a + rnd, 16)
                hi = (b + rnd) & hmask
                rbb[r, pl.ds(16 * q, 16)] = lo | hi
        pltpu.sync_copy(rbb, out_hbm.at[pl.ds(ooff, RB)])
        return carry

    lax.fori_loop(0, nrows // RB, readback, 0)


def _tc_scale_body(x_ref, w_ref, degp_ref, g_ref, dinv_ref):
    deg = degp_ref[0:N, 0:1] + degp_ref[NACC:NACC + N, 0:1] + 1.0
    dinv = lax.rsqrt(deg)
    h = jnp.dot(x_ref[...], w_ref[...], preferred_element_type=_f32)
    g_ref[0:N] = h * dinv
    dinv_ref[...] = dinv


def _unpack2(v):
    lo = lax.bitcast_convert_type(lax.shift_left(v, 16), _f32)
    hi = lax.bitcast_convert_type(v & jnp.int32(-65536), _f32)
    return lo, hi


def _tc_mid_body(s1_ref, dinv_ref, b1p_ref, g2_ref):
    p0l, p0h = _unpack2(s1_ref[0:N])
    p1l, p1h = _unpack2(s1_ref[N:2 * N])
    agg = jnp.concatenate([p0l + p1l, p0h + p1h], axis=1)
    dinv = dinv_ref[...]
    out1 = jnp.maximum(agg * dinv + b1p_ref[...], 0.0)
    g2_ref[0:N] = out1 * dinv


def _tc_out_body(s2_ref, dinv_ref, w2pp_ref, b2_ref, o_ref):
    p0l, p0h = _unpack2(s2_ref[0:N])
    p1l, p1h = _unpack2(s2_ref[N:2 * N])
    agg = jnp.concatenate([p0l + p1l, p0h + p1h], axis=1)
    out = jnp.dot(agg * dinv_ref[...], w2pp_ref[...],
                  preferred_element_type=_f32)
    o_ref[...] = out + b2_ref[...]


@jax.jit
def kernel(x, edge_index, W1, b1, W2, b2):
    src_e = edge_index[0].astype(_i32)
    dst_e = edge_index[1].astype(_i32)
    dst = jnp.concatenate(
        [dst_e, jnp.full((ED - E,), PADROW, _i32)]
    ).reshape(NW, NCHUNK, CH)

    loop = jnp.arange(N, dtype=_i32)
    src1 = jnp.concatenate(
        [src_e, loop, jnp.zeros((E1 - E - N,), _i32)]
    ).reshape(NW, NCHUNK1, CH)
    dst1 = jnp.concatenate(
        [dst_e, loop, jnp.full((E1 - E - N,), PADROW, _i32)]
    ).reshape(NW, NCHUNK1, CH)

    degp = _sc_degree(dst)

    g1, dinv = pl.pallas_call(
        _tc_scale_body,
        out_shape=(
            jax.ShapeDtypeStruct((2 * N, D1), _f32),
            jax.ShapeDtypeStruct((N, 1), _f32),
        ),
    )(x, W1, degp)

    s1 = _sc_agg1(src1, dst1, g1)

    b1p = b1[jnp.asarray(PERM)].reshape(1, D1)
    W2p = jnp.zeros((D1, DP), _f32).at[:, :DO].set(W2)
    W2pp = W2p[jnp.asarray(PERM[PERM]), :]
    b2p = jnp.zeros((1, DP), _f32).at[0, :DO].set(b2)

    g2 = pl.pallas_call(
        _tc_mid_body,
        out_shape=jax.ShapeDtypeStruct((2 * N, D1), _f32),
    )(s1, dinv, b1p)

    s2 = _sc_agg1(src1, dst1, g2)

    outp = pl.pallas_call(
        _tc_out_body,
        out_shape=jax.ShapeDtypeStruct((N, DP), _f32),
    )(s2, dinv, W2pp, b2p)

    return outp[:, :DO]
